# baseline (speedup 1.0000x reference)
import jax
import jax.numpy as jnp
import numpy as np
from jax import lax
from jax.experimental import pallas as pl
from jax.experimental.pallas import tpu as pltpu
from jax.experimental.pallas import tpu_sc as plsc

BATCH = 16384
HIST = 200
EMB_DIM = 128
HID = 32
PACK = EMB_DIM // HID

NC, NS = 2, 16
NW = NC * NS
B_PER_W = BATCH // NW
SLAB = 8
N_SLAB = B_PER_W // SLAB
IDX_PER_SLAB = SLAB * HIST
IDX_PAD = 208
PROJ_BLK = 2000

_COL_PERM = np.arange(HID).reshape(2, HID // 2).T.reshape(-1)


def _proj_body(e0_ref, e1_ref, e2_ref, e3_ref, w_ref, out_ref):
    parts = [
        (lax.dot_general(e_ref[...], w_ref[...], (((1,), (1,)), ((), ())),
                         preferred_element_type=jnp.float32)
         * (1.0 / HIST)).astype(jnp.bfloat16)
        for e_ref in (e0_ref, e1_ref, e2_ref, e3_ref)
    ]
    out_ref[...] = jnp.concatenate(parts, axis=1)


def _make_proj(n_vocab):
    quarter_blocks = n_vocab // PACK // PROJ_BLK
    return pl.pallas_call(
        _proj_body,
        grid=(quarter_blocks,),
        in_specs=[
            pl.BlockSpec((PROJ_BLK, EMB_DIM),
                         lambda i, k=k: (i + k * quarter_blocks, 0))
            for k in range(PACK)
        ] + [pl.BlockSpec((HID, EMB_DIM), lambda i: (0, 0))],
        out_specs=pl.BlockSpec((PROJ_BLK, EMB_DIM), lambda i: (i, 0)),
        out_shape=jax.ShapeDtypeStruct((n_vocab // PACK, EMB_DIM),
                                       jnp.bfloat16),
        compiler_params=pltpu.CompilerParams(
            dimension_semantics=("arbitrary",)),
    )


def _pool_body(proj_hbm, x_hbm, out_hbm, idx_v, rows_v, out_v,
               gsem0, gsem1, isem0, isem1):
    cid = lax.axis_index("c")
    sid = lax.axis_index("s")
    wid = sid * NC + cid
    out_row0 = wid * B_PER_W
    gsems = (gsem0, gsem1)
    isems = (isem0, isem1)
    quarter = 250000

    def idx_copy(slab, ibuf):
        return pltpu.make_async_copy(
            x_hbm.at[pl.ds(out_row0 + slab * SLAB, SLAB)],
            idx_v.at[ibuf, :, pl.ds(0, HIST)], isems[ibuf])

    def remap_indices(ibuf):
        for g in range(SLAB):
            @pl.loop(0, IDX_PAD // 16, unroll=4)
            def _(i):
                v = idx_v[ibuf, g, pl.ds(i * 16, 16)]
                k = ((v >= quarter).astype(jnp.int32)
                     + (v >= 2 * quarter).astype(jnp.int32)
                     + (v >= 3 * quarter).astype(jnp.int32))
                idx_v[ibuf, g, pl.ds(i * 16, 16)] = (
                    (v << 2) - (PACK * quarter - 1) * k)

    def gathers(buf, ibuf):
        for g in range(SLAB):
            for off, ln in ((0, 128), (128, 72)):
                yield (proj_hbm.at[idx_v.at[ibuf, g, pl.ds(off, ln)]],
                       rows_v.at[buf, pl.ds(g * HIST + off, ln)],
                       gsems[buf])

    def fire(buf, ibuf):
        for src, dst, sem in gathers(buf, ibuf):
            pltpu.async_copy(src, dst, sem)

    def drain(buf, ibuf):
        for src, dst, sem in gathers(buf, ibuf):
            pltpu.make_async_copy(src, dst, sem).wait()

    def reduce_slab(slab, buf):
        for g in range(SLAB):
            base = g * HIST
            zero = jnp.zeros((16,), jnp.float32)

            def body(l, carry):
                a0, a1, b0, b1 = carry
                r0, r1 = plsc.unpack(rows_v[buf, base + 2 * l, ...],
                                     format=plsc.PackFormat.INTERLEAVED)
                s0, s1 = plsc.unpack(rows_v[buf, base + 2 * l + 1, ...],
                                     format=plsc.PackFormat.INTERLEAVED)
                return a0 + r0, a1 + r1, b0 + s0, b1 + s1

            a0, a1, b0, b1 = lax.fori_loop(
                0, HIST // 2, body, (zero, zero, zero, zero), unroll=4)
            q = 32 * (g % PACK)
            out_v[g // PACK, pl.ds(q, 16)] = a0 + b0
            out_v[g // PACK, pl.ds(q + 16, 16)] = a1 + b1
        pltpu.sync_copy(
            out_v,
            out_hbm.at[pl.ds((out_row0 + slab * SLAB) // PACK, SLAB // PACK)])

    def start_idx(slab, ibuf):
        pltpu.async_copy(
            x_hbm.at[pl.ds(out_row0 + slab * SLAB, SLAB)],
            idx_v.at[ibuf, :, pl.ds(0, HIST)], isems[ibuf])

    start_idx(0, 0)
    idx_copy(0, 0).wait()
    remap_indices(0)
    fire(0, 0)
    start_idx(1, 1)

    @pl.loop(0, N_SLAB, step=2)
    def _(s):
        for b in (0, 1):
            slab = s + b

            @pl.when(slab + 1 < N_SLAB)
            def _():
                idx_copy(slab + 1, 1 - b).wait()
                remap_indices(1 - b)
                fire(1 - b, 1 - b)

            drain(b, b)

            @pl.when(slab + 2 < N_SLAB)
            def _():
                start_idx(slab + 2, b)

            reduce_slab(slab, b)


_pool_kernel = pl.kernel(
    _pool_body,
    out_type=jax.ShapeDtypeStruct((BATCH // PACK, EMB_DIM), jnp.float32),
    mesh=plsc.VectorSubcoreMesh(
        core_axis_name="c", subcore_axis_name="s",
        num_cores=NC, num_subcores=NS),
    scratch_types=[
        pltpu.VMEM((2, SLAB, IDX_PAD), jnp.int32),
        pltpu.VMEM((2, IDX_PER_SLAB, HID), jnp.bfloat16),
        pltpu.VMEM((SLAB // PACK, EMB_DIM), jnp.float32),
        pltpu.SemaphoreType.DMA,
        pltpu.SemaphoreType.DMA,
        pltpu.SemaphoreType.DMA,
        pltpu.SemaphoreType.DMA,
    ],
    compiler_params=pltpu.CompilerParams(use_tc_tiling_on_sc=False,
                                         needs_layout_passes=False),
)


def _head_body(s_ref, fcb_ref, w2_ref, b2_ref, o_ref):
    h = jnp.maximum(s_ref[...] + fcb_ref[...], 0.0)
    hw = h * w2_ref[...]
    lanes = lax.broadcasted_iota(jnp.int32, (EMB_DIM, PACK), 0)
    cols = lax.broadcasted_iota(jnp.int32, (EMB_DIM, PACK), 1)
    gmat = (lanes // HID == cols).astype(jnp.float32)
    z = lax.dot_general(hw, gmat, (((1,), (0,)), ((), ())),
                        preferred_element_type=jnp.float32)
    o_ref[...] = jax.nn.sigmoid(z + b2_ref[0, 0])


_HEAD_BLK = 2048
_head = pl.pallas_call(
    _head_body,
    grid=(BATCH // PACK // _HEAD_BLK,),
    in_specs=[
        pl.BlockSpec((_HEAD_BLK, EMB_DIM), lambda i: (i, 0)),
        pl.BlockSpec((1, EMB_DIM), lambda i: (0, 0)),
        pl.BlockSpec((1, EMB_DIM), lambda i: (0, 0)),
        pl.BlockSpec((1, 1), lambda i: (0, 0)),
    ],
    out_specs=pl.BlockSpec((_HEAD_BLK, PACK), lambda i: (i, 0)),
    out_shape=jax.ShapeDtypeStruct((BATCH // PACK, PACK), jnp.float32),
)


def kernel(x, emb, fc_w, fc_b, fc2_w, fc2_b):
    n_vocab = emb.shape[0]
    proj = _make_proj(n_vocab)(emb, emb, emb, emb, fc_w[_COL_PERM, :])
    table = jnp.reshape(proj, (n_vocab, HID))
    sums = _pool_kernel(table, x.astype(jnp.int32))
    fcb4 = jnp.tile(jnp.reshape(fc_b, (1, HID)), (1, PACK))
    w24 = jnp.tile(fc2_w, (1, PACK))
    out4 = _head(sums, fcb4, w24, jnp.reshape(fc2_b, (1, 1)))
    return jnp.reshape(out4, (BATCH, 1))


# speedup vs baseline: 1.3202x; 1.3202x over previous
import jax
import jax.numpy as jnp
import numpy as np
from jax import lax
from jax.experimental import pallas as pl
from jax.experimental.pallas import tpu as pltpu
from jax.experimental.pallas import tpu_sc as plsc

BATCH = 16384
HIST = 200
EMB_DIM = 128
HID = 32
PACK = EMB_DIM // HID

NC, NS = 2, 16
NW = NC * NS
B_PER_W = BATCH // NW
SLAB = 8
N_SLAB = B_PER_W // SLAB
IDX_PER_SLAB = SLAB * HIST
IDX_PAD = 208
PROJ_BLK = 2000

_COL_PERM = np.arange(HID).reshape(2, HID // 2).T.reshape(-1)


def _proj_body(e0_ref, e1_ref, e2_ref, e3_ref, w_ref, out_ref):
    parts = [
        lax.dot_general(e_ref[...], w_ref[...], (((1,), (1,)), ((), ())),
                        preferred_element_type=jnp.float32) * (1.0 / HIST)
        for e_ref in (e0_ref, e1_ref, e2_ref, e3_ref)
    ]
    out_ref[...] = jnp.concatenate(parts, axis=1)


def _make_proj(n_vocab):
    quarter_blocks = n_vocab // PACK // PROJ_BLK
    return pl.pallas_call(
        _proj_body,
        grid=(quarter_blocks,),
        in_specs=[
            pl.BlockSpec((PROJ_BLK, EMB_DIM),
                         lambda i, k=k: (i + k * quarter_blocks, 0))
            for k in range(PACK)
        ] + [pl.BlockSpec((HID, EMB_DIM), lambda i: (0, 0))],
        out_specs=pl.BlockSpec((PROJ_BLK, EMB_DIM), lambda i: (i, 0)),
        out_shape=jax.ShapeDtypeStruct((n_vocab // PACK, EMB_DIM),
                                       jnp.float32),
        compiler_params=pltpu.CompilerParams(
            dimension_semantics=("arbitrary",)),
    )


def _pool_body(proj_hbm, x_hbm, out_hbm, idx_v, rows_v, out_v,
               gsem0, gsem1, isem0, isem1):
    cid = lax.axis_index("c")
    sid = lax.axis_index("s")
    wid = sid * NC + cid
    out_row0 = wid * B_PER_W
    gsems = (gsem0, gsem1)
    isems = (isem0, isem1)
    quarter = 250000

    def idx_copy(slab, ibuf):
        return pltpu.make_async_copy(
            x_hbm.at[pl.ds(out_row0 + slab * SLAB, SLAB)],
            idx_v.at[ibuf, :, pl.ds(0, HIST)], isems[ibuf])

    def remap_indices(ibuf):
        for g in range(SLAB):
            @pl.loop(0, IDX_PAD // 16, unroll=4)
            def _(i):
                v = idx_v[ibuf, g, pl.ds(i * 16, 16)]
                k = ((v >= quarter).astype(jnp.int32)
                     + (v >= 2 * quarter).astype(jnp.int32)
                     + (v >= 3 * quarter).astype(jnp.int32))
                idx_v[ibuf, g, pl.ds(i * 16, 16)] = (
                    (v << 2) - (PACK * quarter - 1) * k)

    def gathers(buf, ibuf):
        for g in range(SLAB):
            for off, ln in ((0, 128), (128, 72)):
                yield (proj_hbm.at[idx_v.at[ibuf, g, pl.ds(off, ln)]],
                       rows_v.at[buf, pl.ds(g * HIST + off, ln)],
                       gsems[buf])

    def fire(buf, ibuf):
        for src, dst, sem in gathers(buf, ibuf):
            pltpu.async_copy(src, dst, sem)

    def drain(buf, ibuf):
        for src, dst, sem in gathers(buf, ibuf):
            pltpu.make_async_copy(src, dst, sem).wait()

    def reduce_slab(slab, buf):
        for g in range(SLAB):
            base = g * HIST
            zero = jnp.zeros((16,), jnp.float32)

            def body(l, carry):
                a0, a1, b0, b1 = carry
                a0 = a0 + rows_v[buf, base + 2 * l, pl.ds(0, 16)]
                a1 = a1 + rows_v[buf, base + 2 * l, pl.ds(16, 16)]
                b0 = b0 + rows_v[buf, base + 2 * l + 1, pl.ds(0, 16)]
                b1 = b1 + rows_v[buf, base + 2 * l + 1, pl.ds(16, 16)]
                return a0, a1, b0, b1

            a0, a1, b0, b1 = lax.fori_loop(
                0, HIST // 2, body, (zero, zero, zero, zero), unroll=4)
            q = 32 * (g % PACK)
            out_v[g // PACK, pl.ds(q, 16)] = a0 + b0
            out_v[g // PACK, pl.ds(q + 16, 16)] = a1 + b1
        pltpu.sync_copy(
            out_v,
            out_hbm.at[pl.ds((out_row0 + slab * SLAB) // PACK, SLAB // PACK)])

    def start_idx(slab, ibuf):
        pltpu.async_copy(
            x_hbm.at[pl.ds(out_row0 + slab * SLAB, SLAB)],
            idx_v.at[ibuf, :, pl.ds(0, HIST)], isems[ibuf])

    start_idx(0, 0)
    idx_copy(0, 0).wait()
    remap_indices(0)
    fire(0, 0)
    start_idx(1, 1)

    @pl.loop(0, N_SLAB, step=2)
    def _(s):
        for b in (0, 1):
            slab = s + b

            @pl.when(slab + 1 < N_SLAB)
            def _():
                idx_copy(slab + 1, 1 - b).wait()
                remap_indices(1 - b)
                fire(1 - b, 1 - b)

            drain(b, b)

            @pl.when(slab + 2 < N_SLAB)
            def _():
                start_idx(slab + 2, b)

            reduce_slab(slab, b)


_pool_kernel = pl.kernel(
    _pool_body,
    out_type=jax.ShapeDtypeStruct((BATCH // PACK, EMB_DIM), jnp.float32),
    mesh=plsc.VectorSubcoreMesh(
        core_axis_name="c", subcore_axis_name="s",
        num_cores=NC, num_subcores=NS),
    scratch_types=[
        pltpu.VMEM((2, SLAB, IDX_PAD), jnp.int32),
        pltpu.VMEM((2, IDX_PER_SLAB, HID), jnp.float32),
        pltpu.VMEM((SLAB // PACK, EMB_DIM), jnp.float32),
        pltpu.SemaphoreType.DMA,
        pltpu.SemaphoreType.DMA,
        pltpu.SemaphoreType.DMA,
        pltpu.SemaphoreType.DMA,
    ],
    compiler_params=pltpu.CompilerParams(use_tc_tiling_on_sc=False,
                                         needs_layout_passes=False),
)


def _head_body(s_ref, fcb_ref, w2_ref, b2_ref, o_ref):
    h = jnp.maximum(s_ref[...] + fcb_ref[...], 0.0)
    hw = h * w2_ref[...]
    lanes = lax.broadcasted_iota(jnp.int32, (EMB_DIM, PACK), 0)
    cols = lax.broadcasted_iota(jnp.int32, (EMB_DIM, PACK), 1)
    gmat = (lanes // HID == cols).astype(jnp.float32)
    z = lax.dot_general(hw, gmat, (((1,), (0,)), ((), ())),
                        preferred_element_type=jnp.float32)
    o_ref[...] = jax.nn.sigmoid(z + b2_ref[0, 0])


_HEAD_BLK = 2048
_head = pl.pallas_call(
    _head_body,
    grid=(BATCH // PACK // _HEAD_BLK,),
    in_specs=[
        pl.BlockSpec((_HEAD_BLK, EMB_DIM), lambda i: (i, 0)),
        pl.BlockSpec((1, EMB_DIM), lambda i: (0, 0)),
        pl.BlockSpec((1, EMB_DIM), lambda i: (0, 0)),
        pl.BlockSpec((1, 1), lambda i: (0, 0)),
    ],
    out_specs=pl.BlockSpec((_HEAD_BLK, PACK), lambda i: (i, 0)),
    out_shape=jax.ShapeDtypeStruct((BATCH // PACK, PACK), jnp.float32),
)


def kernel(x, emb, fc_w, fc_b, fc2_w, fc2_b):
    n_vocab = emb.shape[0]
    proj = _make_proj(n_vocab)(emb, emb, emb, emb, fc_w)
    table = jnp.reshape(proj, (n_vocab, HID))
    sums = _pool_kernel(table, x.astype(jnp.int32))
    fcb4 = jnp.tile(jnp.reshape(fc_b, (1, HID)), (1, PACK))
    w24 = jnp.tile(fc2_w, (1, PACK))
    out4 = _head(sums, fcb4, w24, jnp.reshape(fc2_b, (1, 1)))
    return jnp.reshape(out4, (BATCH, 1))


# speedup vs baseline: 1.3609x; 1.0308x over previous
import jax
import jax.numpy as jnp
import numpy as np
from jax import lax
from jax.experimental import pallas as pl
from jax.experimental.pallas import tpu as pltpu
from jax.experimental.pallas import tpu_sc as plsc

BATCH = 16384
HIST = 200
EMB_DIM = 128
HID = 32
PACK = EMB_DIM // HID

NC, NS = 2, 16
NW = NC * NS
B_PER_W = BATCH // NW
SLAB = 8
N_SLAB = B_PER_W // SLAB
IDX_PER_SLAB = SLAB * HIST
IDX_PAD = 208
PROJ_BLK = 2000

_COL_PERM = np.arange(HID).reshape(2, HID // 2).T.reshape(-1)


def _proj_body(e0_ref, e1_ref, e2_ref, e3_ref, w_ref, out_ref):
    parts = [
        lax.dot_general(e_ref[...], w_ref[...], (((1,), (1,)), ((), ())),
                        preferred_element_type=jnp.float32) * (1.0 / HIST)
        for e_ref in (e0_ref, e1_ref, e2_ref, e3_ref)
    ]
    out_ref[...] = jnp.concatenate(parts, axis=1)


def _make_proj(n_vocab):
    quarter_blocks = n_vocab // PACK // PROJ_BLK
    return pl.pallas_call(
        _proj_body,
        grid=(quarter_blocks,),
        in_specs=[
            pl.BlockSpec((PROJ_BLK, EMB_DIM),
                         lambda i, k=k: (i + k * quarter_blocks, 0))
            for k in range(PACK)
        ] + [pl.BlockSpec((HID, EMB_DIM), lambda i: (0, 0))],
        out_specs=pl.BlockSpec((PROJ_BLK, EMB_DIM), lambda i: (i, 0)),
        out_shape=jax.ShapeDtypeStruct((n_vocab // PACK, EMB_DIM),
                                       jnp.float32),
        compiler_params=pltpu.CompilerParams(
            dimension_semantics=("arbitrary",)),
    )


def _pool_body(proj_hbm, x_hbm, out_hbm, idx_v, rows_v, out_v,
               gsem0, gsem1, isem0, isem1):
    cid = lax.axis_index("c")
    sid = lax.axis_index("s")
    wid = sid * NC + cid
    out_row0 = wid * B_PER_W
    gsems = (gsem0, gsem1)
    isems = (isem0, isem1)
    quarter = 250000

    idx0 = out_row0 * HIST

    def idx_copy(slab, ibuf):
        return pltpu.make_async_copy(
            x_hbm.at[pl.ds(idx0 + slab * IDX_PER_SLAB, IDX_PER_SLAB)],
            idx_v.at[ibuf], isems[ibuf])

    def remap_indices(ibuf):
        @pl.loop(0, IDX_PER_SLAB // 16, unroll=4)
        def _(i):
            v = idx_v[ibuf, pl.ds(i * 16, 16)]
            k = ((v >= quarter).astype(jnp.int32)
                 + (v >= 2 * quarter).astype(jnp.int32)
                 + (v >= 3 * quarter).astype(jnp.int32))
            idx_v[ibuf, pl.ds(i * 16, 16)] = (v << 2) - (PACK * quarter - 1) * k

    def gathers(buf, ibuf):
        for g in range(SLAB):
            for off, ln in ((0, 128), (128, 72)):
                yield (proj_hbm.at[idx_v.at[ibuf, pl.ds(g * HIST + off, ln)]],
                       rows_v.at[buf, pl.ds(g * HIST + off, ln)],
                       gsems[buf])

    def fire(buf, ibuf):
        for src, dst, sem in gathers(buf, ibuf):
            pltpu.async_copy(src, dst, sem)

    def drain(buf, ibuf):
        for src, dst, sem in gathers(buf, ibuf):
            pltpu.make_async_copy(src, dst, sem).wait()

    def reduce_slab(slab, buf):
        for g in range(SLAB):
            base = g * HIST
            zero = jnp.zeros((16,), jnp.float32)

            def body(l, carry):
                a0, a1 = carry
                a0 = a0 + rows_v[buf, base + l, pl.ds(0, 16)]
                a1 = a1 + rows_v[buf, base + l, pl.ds(16, 16)]
                return a0, a1

            a0, a1 = lax.fori_loop(0, HIST, body, (zero, zero), unroll=8)
            q = 32 * (g % PACK)
            out_v[g // PACK, pl.ds(q, 16)] = a0
            out_v[g // PACK, pl.ds(q + 16, 16)] = a1
        pltpu.sync_copy(
            out_v,
            out_hbm.at[pl.ds((out_row0 + slab * SLAB) // PACK, SLAB // PACK)])

    def start_idx(slab, ibuf):
        pltpu.async_copy(
            x_hbm.at[pl.ds(idx0 + slab * IDX_PER_SLAB, IDX_PER_SLAB)],
            idx_v.at[ibuf], isems[ibuf])

    start_idx(0, 0)
    idx_copy(0, 0).wait()
    remap_indices(0)
    fire(0, 0)
    start_idx(1, 1)

    @pl.loop(0, N_SLAB, step=2)
    def _(s):
        for b in (0, 1):
            slab = s + b

            @pl.when(slab + 1 < N_SLAB)
            def _():
                idx_copy(slab + 1, 1 - b).wait()
                remap_indices(1 - b)
                fire(1 - b, 1 - b)

            drain(b, b)

            @pl.when(slab + 2 < N_SLAB)
            def _():
                start_idx(slab + 2, b)

            reduce_slab(slab, b)


_pool_kernel = pl.kernel(
    _pool_body,
    out_type=jax.ShapeDtypeStruct((BATCH // PACK, EMB_DIM), jnp.float32),
    mesh=plsc.VectorSubcoreMesh(
        core_axis_name="c", subcore_axis_name="s",
        num_cores=NC, num_subcores=NS),
    scratch_types=[
        pltpu.VMEM((2, IDX_PER_SLAB), jnp.int32),
        pltpu.VMEM((2, IDX_PER_SLAB, HID), jnp.float32),
        pltpu.VMEM((SLAB // PACK, EMB_DIM), jnp.float32),
        pltpu.SemaphoreType.DMA,
        pltpu.SemaphoreType.DMA,
        pltpu.SemaphoreType.DMA,
        pltpu.SemaphoreType.DMA,
    ],
    compiler_params=pltpu.CompilerParams(use_tc_tiling_on_sc=False,
                                         needs_layout_passes=False),
)


def _head_body(s_ref, fcb_ref, w2_ref, b2_ref, o_ref):
    h = jnp.maximum(s_ref[...] + fcb_ref[...], 0.0)
    hw = h * w2_ref[...]
    lanes = lax.broadcasted_iota(jnp.int32, (EMB_DIM, PACK), 0)
    cols = lax.broadcasted_iota(jnp.int32, (EMB_DIM, PACK), 1)
    gmat = (lanes // HID == cols).astype(jnp.float32)
    z = lax.dot_general(hw, gmat, (((1,), (0,)), ((), ())),
                        preferred_element_type=jnp.float32)
    o_ref[...] = jax.nn.sigmoid(z + b2_ref[0, 0])


_HEAD_BLK = 2048
_head = pl.pallas_call(
    _head_body,
    grid=(BATCH // PACK // _HEAD_BLK,),
    in_specs=[
        pl.BlockSpec((_HEAD_BLK, EMB_DIM), lambda i: (i, 0)),
        pl.BlockSpec((1, EMB_DIM), lambda i: (0, 0)),
        pl.BlockSpec((1, EMB_DIM), lambda i: (0, 0)),
        pl.BlockSpec((1, 1), lambda i: (0, 0)),
    ],
    out_specs=pl.BlockSpec((_HEAD_BLK, PACK), lambda i: (i, 0)),
    out_shape=jax.ShapeDtypeStruct((BATCH // PACK, PACK), jnp.float32),
)


def kernel(x, emb, fc_w, fc_b, fc2_w, fc2_b):
    n_vocab = emb.shape[0]
    proj = _make_proj(n_vocab)(emb, emb, emb, emb, fc_w)
    table = jnp.reshape(proj, (n_vocab, HID))
    sums = _pool_kernel(table, jnp.reshape(x.astype(jnp.int32), (-1,)))
    fcb4 = jnp.tile(jnp.reshape(fc_b, (1, HID)), (1, PACK))
    w24 = jnp.tile(fc2_w, (1, PACK))
    out4 = _head(sums, fcb4, w24, jnp.reshape(fc2_b, (1, 1)))
    return jnp.reshape(out4, (BATCH, 1))


# speedup vs baseline: 1.3620x; 1.0008x over previous
import jax
import jax.numpy as jnp
import numpy as np
from jax import lax
from jax.experimental import pallas as pl
from jax.experimental.pallas import tpu as pltpu
from jax.experimental.pallas import tpu_sc as plsc

BATCH = 16384
HIST = 200
EMB_DIM = 128
HID = 32
PACK = EMB_DIM // HID

NC, NS = 2, 16
NW = NC * NS
B_PER_W = BATCH // NW
SLAB = 8
N_SLAB = B_PER_W // SLAB
IDX_PER_SLAB = SLAB * HIST
IDX_PAD = 208
PROJ_BLK = 2000

_COL_PERM = np.arange(HID).reshape(2, HID // 2).T.reshape(-1)


def _proj_body(e0_ref, e1_ref, e2_ref, e3_ref, w_ref, out_ref):
    parts = [
        lax.dot_general(e_ref[...], w_ref[...], (((1,), (1,)), ((), ())),
                        preferred_element_type=jnp.float32) * (1.0 / HIST)
        for e_ref in (e0_ref, e1_ref, e2_ref, e3_ref)
    ]
    out_ref[...] = jnp.concatenate(parts, axis=1)


def _make_proj(n_vocab):
    quarter_blocks = n_vocab // PACK // PROJ_BLK
    return pl.pallas_call(
        _proj_body,
        grid=(quarter_blocks,),
        in_specs=[
            pl.BlockSpec((PROJ_BLK, EMB_DIM),
                         lambda i, k=k: (i + k * quarter_blocks, 0))
            for k in range(PACK)
        ] + [pl.BlockSpec((HID, EMB_DIM), lambda i: (0, 0))],
        out_specs=pl.BlockSpec((PROJ_BLK, EMB_DIM), lambda i: (i, 0)),
        out_shape=jax.ShapeDtypeStruct((n_vocab // PACK, EMB_DIM),
                                       jnp.float32),
        compiler_params=pltpu.CompilerParams(
            dimension_semantics=("arbitrary",)),
    )


def _pool_body(proj_hbm, x_hbm, out_hbm, idx_v, rows_v, out_v,
               gsem0, gsem1, isem0, isem1):
    cid = lax.axis_index("c")
    sid = lax.axis_index("s")
    wid = sid * NC + cid
    out_row0 = wid * B_PER_W
    gsems = (gsem0, gsem1)
    isems = (isem0, isem1)
    quarter = 250000

    def idx_copies(slab, ibuf):
        for g in range(SLAB):
            yield (x_hbm.at[out_row0 + slab * SLAB + g],
                   idx_v.at[ibuf, pl.ds(g * HIST, HIST)], isems[ibuf])

    def idx_wait(slab, ibuf):
        for src, dst, sem in idx_copies(slab, ibuf):
            pltpu.make_async_copy(src, dst, sem).wait()

    def remap_indices(ibuf):
        @pl.loop(0, IDX_PER_SLAB // 16, unroll=4)
        def _(i):
            v = idx_v[ibuf, pl.ds(i * 16, 16)]
            k = ((v >= quarter).astype(jnp.int32)
                 + (v >= 2 * quarter).astype(jnp.int32)
                 + (v >= 3 * quarter).astype(jnp.int32))
            idx_v[ibuf, pl.ds(i * 16, 16)] = (v << 2) - (PACK * quarter - 1) * k

    def gathers(buf, ibuf):
        for g in range(SLAB):
            for off, ln in ((0, 128), (128, 72)):
                yield (proj_hbm.at[idx_v.at[ibuf, pl.ds(g * HIST + off, ln)]],
                       rows_v.at[buf, pl.ds(g * HIST + off, ln)],
                       gsems[buf])

    def fire(buf, ibuf):
        for src, dst, sem in gathers(buf, ibuf):
            pltpu.async_copy(src, dst, sem)

    def drain(buf, ibuf):
        for src, dst, sem in gathers(buf, ibuf):
            pltpu.make_async_copy(src, dst, sem).wait()

    def reduce_slab(slab, buf):
        for g in range(SLAB):
            base = g * HIST
            zero = jnp.zeros((16,), jnp.float32)

            def body(l, carry):
                a0, a1 = carry
                a0 = a0 + rows_v[buf, base + l, pl.ds(0, 16)]
                a1 = a1 + rows_v[buf, base + l, pl.ds(16, 16)]
                return a0, a1

            a0, a1 = lax.fori_loop(0, HIST, body, (zero, zero), unroll=8)
            q = 32 * (g % PACK)
            out_v[g // PACK, pl.ds(q, 16)] = a0
            out_v[g // PACK, pl.ds(q + 16, 16)] = a1
        pltpu.sync_copy(
            out_v,
            out_hbm.at[pl.ds((out_row0 + slab * SLAB) // PACK, SLAB // PACK)])

    def start_idx(slab, ibuf):
        for src, dst, sem in idx_copies(slab, ibuf):
            pltpu.async_copy(src, dst, sem)

    start_idx(0, 0)
    idx_wait(0, 0)
    remap_indices(0)
    fire(0, 0)
    start_idx(1, 1)

    @pl.loop(0, N_SLAB, step=2)
    def _(s):
        for b in (0, 1):
            slab = s + b

            @pl.when(slab + 1 < N_SLAB)
            def _():
                idx_wait(slab + 1, 1 - b)
                remap_indices(1 - b)
                fire(1 - b, 1 - b)

            drain(b, b)

            @pl.when(slab + 2 < N_SLAB)
            def _():
                start_idx(slab + 2, b)

            reduce_slab(slab, b)


_pool_kernel = pl.kernel(
    _pool_body,
    out_type=jax.ShapeDtypeStruct((BATCH // PACK, EMB_DIM), jnp.float32),
    mesh=plsc.VectorSubcoreMesh(
        core_axis_name="c", subcore_axis_name="s",
        num_cores=NC, num_subcores=NS),
    scratch_types=[
        pltpu.VMEM((2, IDX_PER_SLAB), jnp.int32),
        pltpu.VMEM((2, IDX_PER_SLAB, HID), jnp.float32),
        pltpu.VMEM((SLAB // PACK, EMB_DIM), jnp.float32),
        pltpu.SemaphoreType.DMA,
        pltpu.SemaphoreType.DMA,
        pltpu.SemaphoreType.DMA,
        pltpu.SemaphoreType.DMA,
    ],
    compiler_params=pltpu.CompilerParams(use_tc_tiling_on_sc=False,
                                         needs_layout_passes=False),
)


def _head_body(s_ref, fcb_ref, w2_ref, b2_ref, o_ref):
    h = jnp.maximum(s_ref[...] + fcb_ref[...], 0.0)
    hw = h * w2_ref[...]
    lanes = lax.broadcasted_iota(jnp.int32, (EMB_DIM, PACK), 0)
    cols = lax.broadcasted_iota(jnp.int32, (EMB_DIM, PACK), 1)
    gmat = (lanes // HID == cols).astype(jnp.float32)
    z = lax.dot_general(hw, gmat, (((1,), (0,)), ((), ())),
                        preferred_element_type=jnp.float32)
    o_ref[...] = jax.nn.sigmoid(z + b2_ref[0, 0])


_HEAD_BLK = 2048
_head = pl.pallas_call(
    _head_body,
    grid=(BATCH // PACK // _HEAD_BLK,),
    in_specs=[
        pl.BlockSpec((_HEAD_BLK, EMB_DIM), lambda i: (i, 0)),
        pl.BlockSpec((1, EMB_DIM), lambda i: (0, 0)),
        pl.BlockSpec((1, EMB_DIM), lambda i: (0, 0)),
        pl.BlockSpec((1, 1), lambda i: (0, 0)),
    ],
    out_specs=pl.BlockSpec((_HEAD_BLK, PACK), lambda i: (i, 0)),
    out_shape=jax.ShapeDtypeStruct((BATCH // PACK, PACK), jnp.float32),
)


def kernel(x, emb, fc_w, fc_b, fc2_w, fc2_b):
    n_vocab = emb.shape[0]
    proj = _make_proj(n_vocab)(emb, emb, emb, emb, fc_w)
    table = jnp.reshape(proj, (n_vocab, HID))
    sums = _pool_kernel(table, x.astype(jnp.int32))
    fcb4 = jnp.tile(jnp.reshape(fc_b, (1, HID)), (1, PACK))
    w24 = jnp.tile(fc2_w, (1, PACK))
    out4 = _head(sums, fcb4, w24, jnp.reshape(fc2_b, (1, 1)))
    return jnp.reshape(out4, (BATCH, 1))
